# baseline (speedup 1.0000x reference)
import functools

import jax
import jax.numpy as jnp
from jax.experimental import pallas as pl
from jax.experimental.pallas import tpu as pltpu

_BN_EPS = 1e-5
_BN1 = 8
_BN2 = 4


def _conv_stats_kernel(x_ref, w_ref, u_ref, sum_ref, sq_ref):
    w = w_ref[...]
    s_acc = None
    q_acc = None
    for i in range(_BN1):
        xb = x_ref[i].astype(jnp.bfloat16)
        u = jnp.dot(w, xb, preferred_element_type=jnp.float32)
        u_ref[i] = u.astype(jnp.bfloat16)
        s_i = jnp.sum(u, axis=-1, keepdims=True)
        q_i = jnp.sum(u * u, axis=-1, keepdims=True)
        s_acc = s_i if s_acc is None else s_acc + s_i
        q_acc = q_i if q_acc is None else q_acc + q_i
    sum_ref[...] = s_acc
    sq_ref[...] = q_acc


def _norm_sigmoid_kernel(inv_count, u_ref, sum_ref, sq_ref, g_ref, b_ref,
                         o_ref):
    sum_u = jnp.sum(sum_ref[...], axis=0)
    sq_u = jnp.sum(sq_ref[...], axis=0)
    mean_u = sum_u * inv_count
    var_u = jnp.maximum(sq_u * inv_count - mean_u * mean_u, 0.0)
    s = g_ref[...] * jax.lax.rsqrt(var_u + _BN_EPS)
    t = b_ref[...] - mean_u * s
    z = u_ref[...].astype(jnp.float32) * s + t
    o_ref[...] = pl.reciprocal(1.0 + jnp.exp(-z), approx=True)


def kernel(x_ncl, weight, bias, gamma, beta):
    del bias
    n, c_in, length = x_ncl.shape
    c_out = weight.shape[0]
    nb1 = n // _BN1
    nb2 = n // _BN2

    x = x_ncl.astype(jnp.float32)
    w = weight[:, :, 0].astype(jnp.bfloat16)

    x_spec = pl.BlockSpec((_BN1, c_in, length), lambda bi: (bi, 0, 0))
    w_spec = pl.BlockSpec((c_out, c_in), lambda bi: (0, 0))
    stat_spec = pl.BlockSpec((None, c_out, 1), lambda bi: (bi, 0, 0))
    u_spec = pl.BlockSpec((_BN1, c_out, length), lambda bi: (bi, 0, 0))
    u2_spec = pl.BlockSpec((_BN2, c_out, length), lambda bi: (bi, 0, 0))

    u_bf16, sum_b, sq_b = pl.pallas_call(
        _conv_stats_kernel,
        out_shape=(jax.ShapeDtypeStruct((n, c_out, length), jnp.bfloat16),
                   jax.ShapeDtypeStruct((nb1, c_out, 1), jnp.float32),
                   jax.ShapeDtypeStruct((nb1, c_out, 1), jnp.float32)),
        grid=(nb1,),
        in_specs=[x_spec, w_spec],
        out_specs=(u_spec, stat_spec, stat_spec),
        compiler_params=pltpu.CompilerParams(
            dimension_semantics=("parallel",)),
    )(x, w)

    inv_count = 1.0 / float(n * length)
    stat_full = pl.BlockSpec((nb1, c_out, 1), lambda bi: (0, 0, 0))
    col_spec = pl.BlockSpec((c_out, 1), lambda bi: (0, 0))
    out = pl.pallas_call(
        functools.partial(_norm_sigmoid_kernel, inv_count),
        out_shape=jax.ShapeDtypeStruct((n, c_out, length), jnp.float32),
        grid=(nb2,),
        in_specs=[u2_spec, stat_full, stat_full, col_spec, col_spec],
        out_specs=pl.BlockSpec((_BN2, c_out, length), lambda bi: (bi, 0, 0)),
        compiler_params=pltpu.CompilerParams(
            dimension_semantics=("parallel",)),
    )(u_bf16, sum_b, sq_b,
      gamma.astype(jnp.float32).reshape(c_out, 1),
      beta.astype(jnp.float32).reshape(c_out, 1))

    return out


# speedup vs baseline: 1.0161x; 1.0161x over previous
import functools

import jax
import jax.numpy as jnp
from jax.experimental import pallas as pl
from jax.experimental.pallas import tpu as pltpu

_BN_EPS = 1e-5
_BN1 = 8
_BN2 = 8


def _conv_stats_kernel(x_ref, w_ref, u_ref, sum_ref, sq_ref):
    w = w_ref[...]
    s_acc = None
    q_acc = None
    for i in range(_BN1):
        xb = x_ref[i].astype(jnp.bfloat16)
        u = jnp.dot(w, xb, preferred_element_type=jnp.float32)
        u_ref[i] = u.astype(jnp.bfloat16)
        s_i = jnp.sum(u, axis=-1, keepdims=True)
        q_i = jnp.sum(u * u, axis=-1, keepdims=True)
        s_acc = s_i if s_acc is None else s_acc + s_i
        q_acc = q_i if q_acc is None else q_acc + q_i
    sum_ref[...] = s_acc
    sq_ref[...] = q_acc


def _norm_sigmoid_kernel(inv_count, u_ref, sum_ref, sq_ref, g_ref, b_ref,
                         o_ref):
    sum_u = jnp.sum(sum_ref[...], axis=0)
    sq_u = jnp.sum(sq_ref[...], axis=0)
    mean_u = sum_u * inv_count
    var_u = jnp.maximum(sq_u * inv_count - mean_u * mean_u, 0.0)
    s = g_ref[...] * jax.lax.rsqrt(var_u + _BN_EPS)
    t = b_ref[...] - mean_u * s
    for i in range(_BN2):
        z = u_ref[i].astype(jnp.float32) * s + t
        o_ref[i] = pl.reciprocal(1.0 + jnp.exp(-z), approx=True)


def kernel(x_ncl, weight, bias, gamma, beta):
    del bias
    n, c_in, length = x_ncl.shape
    c_out = weight.shape[0]
    nb1 = n // _BN1
    nb2 = n // _BN2

    x = x_ncl.astype(jnp.float32)
    w = weight[:, :, 0].astype(jnp.bfloat16)

    x_spec = pl.BlockSpec((_BN1, c_in, length), lambda bi: (bi, 0, 0))
    w_spec = pl.BlockSpec((c_out, c_in), lambda bi: (0, 0))
    stat_spec = pl.BlockSpec((None, c_out, 1), lambda bi: (bi, 0, 0))
    u_spec = pl.BlockSpec((_BN1, c_out, length), lambda bi: (bi, 0, 0))
    u2_spec = pl.BlockSpec((_BN2, c_out, length), lambda bi: (bi, 0, 0))

    u_bf16, sum_b, sq_b = pl.pallas_call(
        _conv_stats_kernel,
        out_shape=(jax.ShapeDtypeStruct((n, c_out, length), jnp.bfloat16),
                   jax.ShapeDtypeStruct((nb1, c_out, 1), jnp.float32),
                   jax.ShapeDtypeStruct((nb1, c_out, 1), jnp.float32)),
        grid=(nb1,),
        in_specs=[x_spec, w_spec],
        out_specs=(u_spec, stat_spec, stat_spec),
        compiler_params=pltpu.CompilerParams(
            dimension_semantics=("parallel",)),
    )(x, w)

    inv_count = 1.0 / float(n * length)
    stat_full = pl.BlockSpec((nb1, c_out, 1), lambda bi: (0, 0, 0))
    col_spec = pl.BlockSpec((c_out, 1), lambda bi: (0, 0))
    out = pl.pallas_call(
        functools.partial(_norm_sigmoid_kernel, inv_count),
        out_shape=jax.ShapeDtypeStruct((n, c_out, length), jnp.float32),
        grid=(nb2,),
        in_specs=[u2_spec, stat_full, stat_full, col_spec, col_spec],
        out_specs=pl.BlockSpec((_BN2, c_out, length), lambda bi: (bi, 0, 0)),
        compiler_params=pltpu.CompilerParams(
            dimension_semantics=("parallel",)),
    )(u_bf16, sum_b, sq_b,
      gamma.astype(jnp.float32).reshape(c_out, 1),
      beta.astype(jnp.float32).reshape(c_out, 1))

    return out
